# Initial kernel scaffold; baseline (speedup 1.0000x reference)
import jax
import jax.numpy as jnp
from jax.experimental import pallas as pl


def kernel(timestamps, time_emb_weight):
    raise NotImplementedError("write your pallas kernel here")


# speedup vs baseline: 3.8085x; 3.8085x over previous
import jax
import jax.numpy as jnp
from jax.experimental import pallas as pl
from jax.experimental.pallas import tpu as pltpu

_BATCH = 4096
_HIST = 200
_BUCKET_LEN = 12
_OUT_DIM = 72
_N = _BATCH * _HIST
_G16 = _N // 16
_WIDE = 16 * _OUT_DIM
_K = 14 * 16
_BS = 1024


def _body(b_ref, ts_ref, ehi_ref, elo_ref, out_ref):
    ts = ts_ref[...]
    t = ts.astype(jnp.float32) / 3600000.0
    acc = jnp.zeros(ts.shape, jnp.int32)
    for i in range(13):
        acc += (t > b_ref[i]).astype(jnp.int32)
    tid = jnp.concatenate([acc] * 14, axis=1)
    dlane = jax.lax.broadcasted_iota(jnp.int32, tid.shape, 1) // 16
    oh = (tid == dlane).astype(jnp.bfloat16)
    dn = (((1,), (0,)), ((), ()))
    o = jax.lax.dot_general(oh, ehi_ref[...], dn,
                            preferred_element_type=jnp.float32)
    o = o + jax.lax.dot_general(oh, elo_ref[...], dn,
                                preferred_element_type=jnp.float32)
    out_ref[...] = o


def kernel(timestamps, time_emb_weight):
    boundaries = jnp.concatenate(
        [jnp.zeros((1,), jnp.float32),
         jnp.exp(jnp.arange(_BUCKET_LEN, dtype=jnp.float32))], axis=0)
    ts = timestamps.reshape(_G16, 16)
    eye = jnp.eye(16, dtype=time_emb_weight.dtype)
    e_full = (time_emb_weight[:, None, None, :]
              * eye[None, :, :, None]).reshape(_K, _WIDE)
    e_hi = e_full.astype(jnp.bfloat16)
    e_lo = (e_full - e_hi.astype(jnp.float32)).astype(jnp.bfloat16)

    out = pl.pallas_call(
        _body,
        grid=(_G16 // _BS,),
        in_specs=[
            pl.BlockSpec(memory_space=pltpu.SMEM),
            pl.BlockSpec((_BS, 16), lambda g: (g, 0)),
            pl.BlockSpec((_K, _WIDE), lambda g: (0, 0)),
            pl.BlockSpec((_K, _WIDE), lambda g: (0, 0)),
        ],
        out_specs=pl.BlockSpec((_BS, _WIDE), lambda g: (g, 0)),
        out_shape=jax.ShapeDtypeStruct((_G16, _WIDE), jnp.float32),
    )(boundaries, ts, e_hi, e_lo)
    return out.reshape(_BATCH, _HIST, _OUT_DIM)


# speedup vs baseline: 6.5493x; 1.7196x over previous
import jax
import jax.numpy as jnp
from jax.experimental import pallas as pl
from jax.experimental.pallas import tpu as pltpu

_BATCH = 4096
_HIST = 200
_BUCKET_LEN = 12
_OUT_DIM = 72
_NB = 14
_B = 64


def _body(b_ref, ts_ref, tab_ref, out_ref):
    ts = ts_ref[...]
    t = ts.astype(jnp.float32) / 3600000.0
    acc = jnp.zeros(ts.shape, jnp.int32)
    for i in range(13):
        acc += (t > b_ref[i]).astype(jnp.int32)
    dio = jax.lax.broadcasted_iota(jnp.int32, (_B, _HIST, _NB), 2)
    oh = (acc[:, :, None] == dio).astype(jnp.bfloat16)
    ohm = oh.reshape(_B * _HIST, _NB)
    dn = (((1,), (0,)), ((), ()))
    r = jax.lax.dot_general(ohm, tab_ref[...], dn,
                            preferred_element_type=jnp.float32)
    out_ref[...] = r.reshape(_B, _HIST, _OUT_DIM)


def kernel(timestamps, time_emb_weight):
    boundaries = jnp.concatenate(
        [jnp.zeros((1,), jnp.float32),
         jnp.exp(jnp.arange(_BUCKET_LEN, dtype=jnp.float32))], axis=0)
    tab = time_emb_weight.astype(jnp.bfloat16)

    return pl.pallas_call(
        _body,
        grid=(_BATCH // _B,),
        in_specs=[
            pl.BlockSpec(memory_space=pltpu.SMEM),
            pl.BlockSpec((_B, _HIST), lambda g: (g, 0)),
            pl.BlockSpec((_NB, _OUT_DIM), lambda g: (0, 0)),
        ],
        out_specs=pl.BlockSpec((_B, _HIST, _OUT_DIM), lambda g: (g, 0, 0)),
        out_shape=jax.ShapeDtypeStruct((_BATCH, _HIST, _OUT_DIM), jnp.float32),
    )(boundaries, timestamps, tab)


# speedup vs baseline: 42.0645x; 6.4228x over previous
import jax
import jax.numpy as jnp
from jax.experimental import pallas as pl
from jax.experimental.pallas import tpu as pltpu

_BATCH = 4096
_HIST = 200
_BUCKET_LEN = 12
_OUT_DIM = 72
_NB = 14
_HB = 8


def _body(b_ref, ts_ref, tab_ref, out_ref):
    ts = ts_ref[...]
    t = ts.astype(jnp.float32) / 3600000.0
    acc = jnp.zeros(ts.shape, jnp.int32)
    for i in range(13):
        acc += (t > b_ref[i]).astype(jnp.int32)
    iot = jax.lax.broadcasted_iota(jnp.int32, (_NB, _BATCH), 0)
    dn = (((1,), (0,)), ((), ()))
    for h in range(_HB):
        oh = (jnp.broadcast_to(acc[h:h + 1, :], (_NB, _BATCH)) == iot)
        r = jax.lax.dot_general(tab_ref[...], oh.astype(jnp.bfloat16), dn,
                                preferred_element_type=jnp.float32)
        out_ref[h] = r


def kernel(timestamps, time_emb_weight):
    boundaries = jnp.concatenate(
        [jnp.zeros((1,), jnp.float32),
         jnp.exp(jnp.arange(_BUCKET_LEN, dtype=jnp.float32))], axis=0)
    tab_t = time_emb_weight.T.astype(jnp.bfloat16)
    ts_t = timestamps.T

    out_t = pl.pallas_call(
        _body,
        grid=(_HIST // _HB,),
        in_specs=[
            pl.BlockSpec(memory_space=pltpu.SMEM),
            pl.BlockSpec((_HB, _BATCH), lambda g: (g, 0)),
            pl.BlockSpec((_OUT_DIM, _NB), lambda g: (0, 0)),
        ],
        out_specs=pl.BlockSpec((_HB, _OUT_DIM, _BATCH), lambda g: (g, 0, 0)),
        out_shape=jax.ShapeDtypeStruct((_HIST, _OUT_DIM, _BATCH), jnp.float32),
    )(boundaries, ts_t, tab_t)
    return jnp.transpose(out_t, (2, 0, 1))
